# baseline (speedup 1.0000x reference)
import jax
import jax.numpy as jnp
from jax import lax
from jax.experimental import pallas as pl
from jax.experimental.pallas import tpu as pltpu
from jax.experimental.pallas import tpu_sc as plsc

_N = 10000
_NP = 10240
_H = 32
_DOUT = 128
_E = 320000
_ETOT = _E + _N
_NW = 32
_CH = 128
_NCH = 81
_EPW = _NCH * _CH
_EPAD = _NW * _EPW
_SENT = _N
_NEG = -1.0e30
_RPT = _NP // 16


def _sc_edge_body(f_hbm, s_hbm, d_hbm, c_hbm, src_hbm, dst_hbm, zf_hbm, zn_hbm,
                  num_o, den_o,
                  s_v, d_v, c_v, src_v, dst_v, w_v,
                  rows_a, rows_b, rows_c, rows_d,
                  srow_a, srow_b, srow_c, srow_d,
                  num_sh, den_sh,
                  sem_ga, sem_gb, sem_gc, sem_gd,
                  sem_sa, sem_sb, sem_sc, sem_sd):
    cid = lax.axis_index("c")
    sid = lax.axis_index("s")
    wid = cid * 16 + sid
    r0 = sid * _RPT

    pltpu.sync_copy(zf_hbm.at[pl.ds(r0, _RPT)], num_sh.at[pl.ds(r0, _RPT)])
    pltpu.sync_copy(zn_hbm.at[pl.ds(r0, _RPT)], den_sh.at[pl.ds(r0, _RPT)])

    pltpu.sync_copy(s_hbm, s_v)
    pltpu.sync_copy(d_hbm, d_v)
    pltpu.sync_copy(c_hbm, c_v)
    pltpu.sync_copy(src_hbm.at[wid], src_v)
    pltpu.sync_copy(dst_hbm.at[wid], dst_v)

    c = c_v[...]

    plsc.subcore_barrier()

    bufs = (rows_a, rows_b, rows_c, rows_d)
    sbufs = (srow_a, srow_b, srow_c, srow_d)
    gsems = (sem_ga, sem_gb, sem_gc, sem_gd)
    ssems = (sem_sa, sem_sb, sem_sc, sem_sd)
    _NB = 4
    _LA = 2

    def _fire_g(j, b):
        pltpu.async_copy(f_hbm.at[src_v.at[j]], bufs[b], gsems[b])

    def _w_chunk(j, b):
        for k in range(_CH // 16):
            sl = pl.ds(k * 16, 16)
            e = (plsc.load_gather(s_v, [src_v[j, sl]]) +
                 plsc.load_gather(d_v, [dst_v[j, sl]]))
            e = jnp.maximum(e, 0.2 * e)
            w_v[j, sl] = jnp.exp(e - c)
        pltpu.async_copy(w_v.at[j], den_sh.at[dst_v.at[j]], ssems[b], add=True)

    def _proc(j, b):
        buf = bufs[b]
        sbuf = sbufs[b]
        pltpu.make_async_copy(f_hbm.at[src_v.at[j]], buf, gsems[b]).wait()

        def _scale(i, c2):
            w16 = w_v[j, pl.ds(i * 16, 16)]
            for u in range(16):
                e = i * 16 + u
                w = w16[u]
                lo, hi = plsc.unpack(buf[e, pl.ds(0, 32)],
                                     format=plsc.PackFormat.INTERLEAVED)
                sbuf[e, pl.ds(0, 16)] = lo * w
                sbuf[e, pl.ds(16, 16)] = hi * w
            return c2
        lax.fori_loop(0, _CH // 16, _scale, 0)
        pltpu.async_copy(sbuf, num_sh.at[dst_v.at[j]], ssems[b], add=True)

    def _wait_sc(j, b):
        pltpu.make_async_copy(w_v.at[j], den_sh.at[dst_v.at[j]], ssems[b]).wait()
        pltpu.make_async_copy(sbufs[b], num_sh.at[dst_v.at[j]], ssems[b]).wait()

    def _oct(i, first):
        j0 = _NB * i
        for b in range(_NB):
            jb = j0 + b
            bf = (b + _LA) % _NB
            if not (first and b < _NB - _LA):
                _wait_sc(jb - (_NB - _LA), bf)
            _fire_g(jnp.minimum(jb + _LA, _NCH - 1), bf)
            _w_chunk(jb, b)
            _proc(jb, b)

    for b in range(_LA):
        _fire_g(b, b)
    _oct(0, True)

    def _oct_body(i, carry):
        _oct(i, False)
        return carry
    lax.fori_loop(1, _NCH // _NB, _oct_body, 0)

    jl = _NCH - 1
    _w_chunk(jl, jl % _NB)
    _proc(jl, jl % _NB)
    for b in range(1, _LA):
        pltpu.make_async_copy(f_hbm.at[src_v.at[jl]], bufs[b], gsems[b]).wait()
    for j in range(jl - _LA, jl + 1):
        _wait_sc(j, j % _NB)

    plsc.subcore_barrier()

    pltpu.sync_copy(num_sh.at[pl.ds(r0, _RPT)], num_o.at[cid, pl.ds(r0, _RPT)])
    pltpu.sync_copy(den_sh.at[pl.ds(r0, _RPT)], den_o.at[cid, pl.ds(r0, _RPT)])


_edge_pass = pl.kernel(
    _sc_edge_body,
    out_type=[jax.ShapeDtypeStruct((2, _NP, _H), jnp.float32),
              jax.ShapeDtypeStruct((2, _NP), jnp.float32)],
    mesh=plsc.VectorSubcoreMesh(core_axis_name="c", subcore_axis_name="s"),
    compiler_params=pltpu.CompilerParams(needs_layout_passes=False,
                                         use_tc_tiling_on_sc=False),
    scratch_types=[
        pltpu.VMEM((_NP,), jnp.float32),
        pltpu.VMEM((_NP,), jnp.float32),
        pltpu.VMEM((16,), jnp.float32),
        pltpu.VMEM((_NCH, _CH), jnp.int32),
        pltpu.VMEM((_NCH, _CH), jnp.int32),
        pltpu.VMEM((_NCH, _CH), jnp.float32),
    ] + [pltpu.VMEM((_CH, _H), jnp.bfloat16)] * 4
      + [pltpu.VMEM((_CH, _H), jnp.float32)] * 4
      + [
        pltpu.VMEM_SHARED((_NP, _H), jnp.float32),
        pltpu.VMEM_SHARED((_NP,), jnp.float32),
    ] + [pltpu.SemaphoreType.DMA] * 8,
)


def _cshift(sd):
    z = jnp.max(sd[:, 0]) + jnp.max(sd[:, 1])
    return jnp.full((1, 1), jnp.maximum(z, 0.2 * z), jnp.float32)


def _tc_stage1(x_ref, w1_ref, a1_ref, f1_ref, sd1_ref, c1_ref):
    f1 = jnp.dot(x_ref[...], w1_ref[...], preferred_element_type=jnp.float32)
    f1_ref[...] = f1
    sd1 = jnp.dot(f1, a1_ref[...], preferred_element_type=jnp.float32)
    sd1_ref[...] = sd1
    c1_ref[...] = _cshift(sd1)


def _tc_stage2(num_ref, den_ref, b1_ref, g_ref, be_ref, w2_ref, a2_ref,
               h1_ref, f2_ref, sd2_ref, c2_ref):
    num = num_ref[0, :_N, :] + num_ref[1, :_N, :]
    den = den_ref[0, :_N] + den_ref[1, :_N]
    h1 = num / (den + 1e-16).reshape(_N, 1) + b1_ref[...]
    h1_ref[...] = h1
    mu = jnp.mean(h1, axis=0, keepdims=True)
    var = jnp.mean((h1 - mu) ** 2, axis=0, keepdims=True)
    t = (h1 - mu) / jnp.sqrt(var + 1e-5) * g_ref[...] + be_ref[...]
    t = jnp.maximum(t, 0.0)
    f2 = jnp.dot(t, w2_ref[...], preferred_element_type=jnp.float32)
    f2_ref[...] = f2
    sd2 = jnp.dot(f2, a2_ref[...], preferred_element_type=jnp.float32)
    sd2_ref[...] = sd2
    c2_ref[...] = _cshift(sd2)


def _tc_stage3(num_ref, den_ref, h1_ref, b2_ref, w3_ref, a3_ref,
               f3_ref, sd3_ref, c3_ref):
    num = num_ref[0, :_N, :] + num_ref[1, :_N, :]
    den = den_ref[0, :_N] + den_ref[1, :_N]
    t = num / (den + 1e-16).reshape(_N, 1) + b2_ref[...]
    h = h1_ref[...] + t
    f3_ref[...] = h
    a3 = jnp.dot(w3_ref[...], a3_ref[...], preferred_element_type=jnp.float32)
    sd3 = jnp.dot(h, a3, preferred_element_type=jnp.float32)
    sd3_ref[...] = sd3
    c3_ref[...] = _cshift(sd3)


def _tc_stage4(num_ref, den_ref, w3_ref, b3_ref, out_ref):
    num = num_ref[0, :_N, :] + num_ref[1, :_N, :]
    den = den_ref[0, :_N] + den_ref[1, :_N]
    agg = num / (den + 1e-16).reshape(_N, 1)
    out_ref[...] = (jnp.dot(agg, w3_ref[...], preferred_element_type=jnp.float32)
                    + b3_ref[...])


def kernel(x, edge_index, W1, a_s1, a_d1, b1, gamma, beta,
           W2, a_s2, a_d2, b2, W3, a_s3, a_d3, b3):
    f32 = jnp.float32
    src = edge_index[0].astype(jnp.int32)
    dst = edge_index[1].astype(jnp.int32)
    loop = jnp.arange(_N, dtype=jnp.int32)
    padi = jnp.full((_EPAD - _ETOT,), _SENT, jnp.int32)
    src3 = jnp.concatenate([src, loop, padi]).reshape(_NW, _NCH, _CH)
    dst3 = jnp.concatenate([dst, loop, padi]).reshape(_NW, _NCH, _CH)
    zf = jnp.zeros((_NP, _H), f32)
    zn = jnp.zeros((_NP,), f32)
    sent = jnp.full((_NP - _N,), _NEG, f32)
    zrow = jnp.zeros((_NP - _N, _H), f32)

    permi = jnp.array([i // 2 + (i % 2) * 16 for i in range(_H)], jnp.int32)

    def pad_fsd(f, sd, c11):
        fp = jnp.concatenate([f, zrow], axis=0)[:, permi].astype(jnp.bfloat16)
        return (fp,
                jnp.concatenate([sd[:, 0], sent]),
                jnp.concatenate([sd[:, 1], sent]),
                jnp.broadcast_to(c11.reshape(()), (16,)))

    a1 = jnp.stack([a_s1, a_d1], axis=1)
    f1, sd1, c1 = pl.pallas_call(_tc_stage1, out_shape=[
        jax.ShapeDtypeStruct((_N, _H), f32),
        jax.ShapeDtypeStruct((_N, 2), f32),
        jax.ShapeDtypeStruct((1, 1), f32)])(x, W1, a1)

    fp, sp, dp, cp = pad_fsd(f1, sd1, c1)
    num1, den1 = _edge_pass(fp, sp, dp, cp, src3, dst3, zf, zn)

    a2 = jnp.stack([a_s2, a_d2], axis=1)
    h1, f2, sd2, c2 = pl.pallas_call(_tc_stage2, out_shape=[
        jax.ShapeDtypeStruct((_N, _H), f32),
        jax.ShapeDtypeStruct((_N, _H), f32),
        jax.ShapeDtypeStruct((_N, 2), f32),
        jax.ShapeDtypeStruct((1, 1), f32)])(
            num1, den1, b1.reshape(1, _H), gamma.reshape(1, _H),
            beta.reshape(1, _H), W2, a2)

    fp, sp, dp, cp = pad_fsd(f2, sd2, c2)
    num2, den2 = _edge_pass(fp, sp, dp, cp, src3, dst3, zf, zn)

    a3 = jnp.stack([a_s3, a_d3], axis=1)
    f3, sd3, c3 = pl.pallas_call(_tc_stage3, out_shape=[
        jax.ShapeDtypeStruct((_N, _H), f32),
        jax.ShapeDtypeStruct((_N, 2), f32),
        jax.ShapeDtypeStruct((1, 1), f32)])(
            num2, den2, h1, b2.reshape(1, _H), W3, a3)

    fp, sp, dp, cp = pad_fsd(f3, sd3, c3)
    num3, den3 = _edge_pass(fp, sp, dp, cp, src3, dst3, zf, zn)

    out = pl.pallas_call(_tc_stage4, out_shape=jax.ShapeDtypeStruct(
        (_N, _DOUT), f32))(num3, den3, W3, b3.reshape(1, _DOUT))
    return out


# speedup vs baseline: 1.2934x; 1.2934x over previous
import jax
import jax.numpy as jnp
from jax import lax
from jax.experimental import pallas as pl
from jax.experimental.pallas import tpu as pltpu
from jax.experimental.pallas import tpu_sc as plsc

_N = 10000
_NP = 10240
_H = 32
_DOUT = 128
_E = 320000
_ETOT = _E + _N
_NW = 32
_CH = 128
_NCH = 81
_EPW = _NCH * _CH
_EPAD = _NW * _EPW
_SENT = _N
_NEG = -1.0e30
_RPT = _NP // 16


def _sc_edge_body(f_hbm, s_hbm, d_hbm, c_hbm, src_hbm, dst_hbm, zf_hbm, zn_hbm,
                  num_o, den_o,
                  s_v, d_v, c_v, src_v, dst_v, w_v,
                  rows_a, rows_b, rows_c, rows_d,
                  num_sh, den_sh,
                  sem_ga, sem_gb, sem_gc, sem_gd,
                  sem_sa, sem_sb, sem_sc, sem_sd):
    cid = lax.axis_index("c")
    sid = lax.axis_index("s")
    wid = cid * 16 + sid
    r0 = sid * _RPT

    pltpu.sync_copy(zf_hbm.at[pl.ds(r0, _RPT)], num_sh.at[pl.ds(r0, _RPT)])
    pltpu.sync_copy(zn_hbm.at[pl.ds(r0, _RPT)], den_sh.at[pl.ds(r0, _RPT)])

    pltpu.sync_copy(s_hbm, s_v)
    pltpu.sync_copy(d_hbm, d_v)
    pltpu.sync_copy(c_hbm, c_v)
    pltpu.sync_copy(src_hbm.at[wid], src_v)
    pltpu.sync_copy(dst_hbm.at[wid], dst_v)

    c = c_v[...]

    plsc.subcore_barrier()

    bufs = (rows_a, rows_b, rows_c, rows_d)
    gsems = (sem_ga, sem_gb, sem_gc, sem_gd)
    ssems = (sem_sa, sem_sb, sem_sc, sem_sd)
    _NB = 4
    _LA = 2

    def _fire_g(j, b):
        pltpu.async_copy(f_hbm.at[src_v.at[j]], bufs[b], gsems[b])

    def _w_chunk(j, b):
        for k in range(_CH // 16):
            sl = pl.ds(k * 16, 16)
            e = (plsc.load_gather(s_v, [src_v[j, sl]]) +
                 plsc.load_gather(d_v, [dst_v[j, sl]]))
            e = jnp.maximum(e, 0.2 * e)
            w_v[j, sl] = jnp.exp(e - c)
        pltpu.async_copy(w_v.at[j], den_sh.at[dst_v.at[j]], ssems[b], add=True)

    def _proc(j, b):
        buf = bufs[b]
        pltpu.make_async_copy(f_hbm.at[src_v.at[j]], buf, gsems[b]).wait()

        def _scale(i, c2):
            w16 = w_v[j, pl.ds(i * 16, 16)]
            for u in range(16):
                e = i * 16 + u
                w = w16[u]
                buf[e, pl.ds(0, 16)] = buf[e, pl.ds(0, 16)] * w
                buf[e, pl.ds(16, 16)] = buf[e, pl.ds(16, 16)] * w
            return c2
        lax.fori_loop(0, _CH // 16, _scale, 0)
        pltpu.async_copy(buf, num_sh.at[dst_v.at[j]], ssems[b], add=True)

    def _wait_sc(j, b):
        pltpu.make_async_copy(w_v.at[j], den_sh.at[dst_v.at[j]], ssems[b]).wait()
        pltpu.make_async_copy(bufs[b], num_sh.at[dst_v.at[j]], ssems[b]).wait()

    def _oct(i, first):
        j0 = _NB * i
        for b in range(_NB):
            jb = j0 + b
            bf = (b + _LA) % _NB
            if not (first and b < _NB - _LA):
                _wait_sc(jb - (_NB - _LA), bf)
            _fire_g(jnp.minimum(jb + _LA, _NCH - 1), bf)
            _w_chunk(jb, b)
            _proc(jb, b)

    for b in range(_LA):
        _fire_g(b, b)
    _oct(0, True)

    def _oct_body(i, carry):
        _oct(i, False)
        return carry
    lax.fori_loop(1, _NCH // _NB, _oct_body, 0)

    jl = _NCH - 1
    _w_chunk(jl, jl % _NB)
    _proc(jl, jl % _NB)
    for b in range(1, _LA):
        pltpu.make_async_copy(f_hbm.at[src_v.at[jl]], bufs[b], gsems[b]).wait()
    for j in range(jl - _LA, jl + 1):
        _wait_sc(j, j % _NB)

    plsc.subcore_barrier()

    pltpu.sync_copy(num_sh.at[pl.ds(r0, _RPT)], num_o.at[cid, pl.ds(r0, _RPT)])
    pltpu.sync_copy(den_sh.at[pl.ds(r0, _RPT)], den_o.at[cid, pl.ds(r0, _RPT)])


_edge_pass = pl.kernel(
    _sc_edge_body,
    out_type=[jax.ShapeDtypeStruct((2, _NP, _H), jnp.float32),
              jax.ShapeDtypeStruct((2, _NP), jnp.float32)],
    mesh=plsc.VectorSubcoreMesh(core_axis_name="c", subcore_axis_name="s"),
    compiler_params=pltpu.CompilerParams(needs_layout_passes=False,
                                         use_tc_tiling_on_sc=False),
    scratch_types=[
        pltpu.VMEM((_NP,), jnp.float32),
        pltpu.VMEM((_NP,), jnp.float32),
        pltpu.VMEM((16,), jnp.float32),
        pltpu.VMEM((_NCH, _CH), jnp.int32),
        pltpu.VMEM((_NCH, _CH), jnp.int32),
        pltpu.VMEM((_NCH, _CH), jnp.float32),
    ] + [pltpu.VMEM((_CH, _H), jnp.float32)] * 4
      + [
        pltpu.VMEM_SHARED((_NP, _H), jnp.float32),
        pltpu.VMEM_SHARED((_NP,), jnp.float32),
    ] + [pltpu.SemaphoreType.DMA] * 8,
)


def _cshift(sd):
    z = jnp.max(sd[:, 0]) + jnp.max(sd[:, 1])
    return jnp.full((1, 1), jnp.maximum(z, 0.2 * z), jnp.float32)


def _tc_stage1(x_ref, w1_ref, a1_ref, f1_ref, sd1_ref, c1_ref):
    f1 = jnp.dot(x_ref[...], w1_ref[...], preferred_element_type=jnp.float32)
    f1_ref[...] = f1
    sd1 = jnp.dot(f1, a1_ref[...], preferred_element_type=jnp.float32)
    sd1_ref[...] = sd1
    c1_ref[...] = _cshift(sd1)


def _tc_stage2(num_ref, den_ref, b1_ref, g_ref, be_ref, w2_ref, a2_ref,
               h1_ref, f2_ref, sd2_ref, c2_ref):
    num = num_ref[0, :_N, :] + num_ref[1, :_N, :]
    den = den_ref[0, :_N] + den_ref[1, :_N]
    h1 = num / (den + 1e-16).reshape(_N, 1) + b1_ref[...]
    h1_ref[...] = h1
    mu = jnp.mean(h1, axis=0, keepdims=True)
    var = jnp.mean((h1 - mu) ** 2, axis=0, keepdims=True)
    t = (h1 - mu) / jnp.sqrt(var + 1e-5) * g_ref[...] + be_ref[...]
    t = jnp.maximum(t, 0.0)
    f2 = jnp.dot(t, w2_ref[...], preferred_element_type=jnp.float32)
    f2_ref[...] = f2
    sd2 = jnp.dot(f2, a2_ref[...], preferred_element_type=jnp.float32)
    sd2_ref[...] = sd2
    c2_ref[...] = _cshift(sd2)


def _tc_stage3(num_ref, den_ref, h1_ref, b2_ref, w3_ref, a3_ref,
               f3_ref, sd3_ref, c3_ref):
    num = num_ref[0, :_N, :] + num_ref[1, :_N, :]
    den = den_ref[0, :_N] + den_ref[1, :_N]
    t = num / (den + 1e-16).reshape(_N, 1) + b2_ref[...]
    h = h1_ref[...] + t
    f3_ref[...] = h
    a3 = jnp.dot(w3_ref[...], a3_ref[...], preferred_element_type=jnp.float32)
    sd3 = jnp.dot(h, a3, preferred_element_type=jnp.float32)
    sd3_ref[...] = sd3
    c3_ref[...] = _cshift(sd3)


def _tc_stage4(num_ref, den_ref, w3_ref, b3_ref, out_ref):
    num = num_ref[0, :_N, :] + num_ref[1, :_N, :]
    den = den_ref[0, :_N] + den_ref[1, :_N]
    agg = num / (den + 1e-16).reshape(_N, 1)
    out_ref[...] = (jnp.dot(agg, w3_ref[...], preferred_element_type=jnp.float32)
                    + b3_ref[...])


def kernel(x, edge_index, W1, a_s1, a_d1, b1, gamma, beta,
           W2, a_s2, a_d2, b2, W3, a_s3, a_d3, b3):
    f32 = jnp.float32
    src = edge_index[0].astype(jnp.int32)
    dst = edge_index[1].astype(jnp.int32)
    loop = jnp.arange(_N, dtype=jnp.int32)
    padi = jnp.full((_EPAD - _ETOT,), _SENT, jnp.int32)
    src3 = jnp.concatenate([src, loop, padi]).reshape(_NW, _NCH, _CH)
    dst3 = jnp.concatenate([dst, loop, padi]).reshape(_NW, _NCH, _CH)
    zf = jnp.zeros((_NP, _H), f32)
    zn = jnp.zeros((_NP,), f32)
    sent = jnp.full((_NP - _N,), _NEG, f32)
    zrow = jnp.zeros((_NP - _N, _H), f32)

    def pad_fsd(f, sd, c11):
        return (jnp.concatenate([f, zrow], axis=0),
                jnp.concatenate([sd[:, 0], sent]),
                jnp.concatenate([sd[:, 1], sent]),
                jnp.broadcast_to(c11.reshape(()), (16,)))

    a1 = jnp.stack([a_s1, a_d1], axis=1)
    f1, sd1, c1 = pl.pallas_call(_tc_stage1, out_shape=[
        jax.ShapeDtypeStruct((_N, _H), f32),
        jax.ShapeDtypeStruct((_N, 2), f32),
        jax.ShapeDtypeStruct((1, 1), f32)])(x, W1, a1)

    fp, sp, dp, cp = pad_fsd(f1, sd1, c1)
    num1, den1 = _edge_pass(fp, sp, dp, cp, src3, dst3, zf, zn)

    a2 = jnp.stack([a_s2, a_d2], axis=1)
    h1, f2, sd2, c2 = pl.pallas_call(_tc_stage2, out_shape=[
        jax.ShapeDtypeStruct((_N, _H), f32),
        jax.ShapeDtypeStruct((_N, _H), f32),
        jax.ShapeDtypeStruct((_N, 2), f32),
        jax.ShapeDtypeStruct((1, 1), f32)])(
            num1, den1, b1.reshape(1, _H), gamma.reshape(1, _H),
            beta.reshape(1, _H), W2, a2)

    fp, sp, dp, cp = pad_fsd(f2, sd2, c2)
    num2, den2 = _edge_pass(fp, sp, dp, cp, src3, dst3, zf, zn)

    a3 = jnp.stack([a_s3, a_d3], axis=1)
    f3, sd3, c3 = pl.pallas_call(_tc_stage3, out_shape=[
        jax.ShapeDtypeStruct((_N, _H), f32),
        jax.ShapeDtypeStruct((_N, 2), f32),
        jax.ShapeDtypeStruct((1, 1), f32)])(
            num2, den2, h1, b2.reshape(1, _H), W3, a3)

    fp, sp, dp, cp = pad_fsd(f3, sd3, c3)
    num3, den3 = _edge_pass(fp, sp, dp, cp, src3, dst3, zf, zn)

    out = pl.pallas_call(_tc_stage4, out_shape=jax.ShapeDtypeStruct(
        (_N, _DOUT), f32))(num3, den3, W3, b3.reshape(1, _DOUT))
    return out
